# baseline (speedup 1.0000x reference)
import jax
import jax.numpy as jnp
from jax import lax
from jax.experimental import pallas as pl
from jax.experimental.pallas import tpu as pltpu
from jax.experimental.pallas import tpu_sc as plsc

_NC = 2
_NS = 16
_NW = _NC * _NS
_B = 128
_MAC = 2
_DC = 64
_DDEG = 16


def _npad(n):
    return ((n + _NS * 128 - 1) // (_NS * 128)) * (_NS * 128)


def _zero_fill(ref, rows, cols):
    z = jnp.zeros((16,), jnp.float32)

    def row(r, carry):
        def col(k, c2):
            ref[r, pl.ds(k * 16, 16)] = z
            return c2
        return lax.fori_loop(0, cols // 16, col, carry)

    lax.fori_loop(0, rows, row, 0)


def _seg_sum_rows(tables, src3d, dst3d, n):
    nt = len(tables)
    nbw = src3d.shape[1]
    nmac = nbw // _MAC
    npad = _npad(n)
    rpt = npad // _NS
    rbr = 128
    assert nbw % _MAC == 0 and rpt % rbr == 0

    mesh = plsc.VectorSubcoreMesh(core_axis_name="c", subcore_axis_name="s")

    def body(*refs):
        table_hbms = refs[:nt]
        src_hbm, dst_hbm = refs[nt], refs[nt + 1]
        out_hbms = refs[nt + 2:2 * nt + 2]
        sidx_v, didx_v, rows_v, rb_v, acc_sh, gsem, ssem = refs[2 * nt + 2:]

        c = lax.axis_index("c")
        s = lax.axis_index("s")
        wid = s * _NC + c

        _zero_fill(rb_v, rbr, _DC)
        for i in range(rpt // rbr):
            pltpu.sync_copy(rb_v, acc_sh.at[pl.ds(s * rpt + i * rbr, rbr)])

        pltpu.sync_copy(src_hbm.at[wid], sidx_v)
        pltpu.sync_copy(dst_hbm.at[wid], didx_v)
        plsc.subcore_barrier()

        for t in range(nt):
            table_hbm = table_hbms[t]

            for k in range(_MAC):
                pltpu.async_copy(
                    table_hbm.at[sidx_v.at[k]], rows_v.at[k], gsem)

            def macro(m, carry):
                mb = lax.rem(m, 2) * _MAC
                for k in range(_MAC):
                    pltpu.make_async_copy(
                        table_hbm.at[sidx_v.at[m * _MAC + k]],
                        rows_v.at[mb + k], gsem).wait()

                @pl.when(m < nmac - 1)
                def _():
                    for k in range(_MAC):
                        pltpu.async_copy(
                            table_hbm.at[sidx_v.at[(m + 1) * _MAC + k]],
                            rows_v.at[(_MAC - mb) + k], gsem)

                descs = [
                    pltpu.async_copy(
                        rows_v.at[mb + k],
                        acc_sh.at[didx_v.at[m * _MAC + k]], ssem, add=True)
                    for k in range(_MAC)
                ]
                for d in descs:
                    d.wait()
                return carry

            lax.fori_loop(0, nmac, macro, 0)
            plsc.subcore_barrier()

            for i in range(rpt // rbr):
                pltpu.sync_copy(acc_sh.at[pl.ds(s * rpt + i * rbr, rbr)],
                                rb_v)
                pltpu.sync_copy(
                    rb_v,
                    out_hbms[t].at[pl.ds(c * npad + s * rpt + i * rbr, rbr)])
                if t < nt - 1:
                    _zero_fill(rb_v, rbr, _DC)
                    pltpu.sync_copy(
                        rb_v, acc_sh.at[pl.ds(s * rpt + i * rbr, rbr)])
            if t < nt - 1:
                plsc.subcore_barrier()

    f = pl.kernel(
        body,
        out_type=[jax.ShapeDtypeStruct((2 * npad, _DC), jnp.float32)
                  for _ in range(nt)],
        mesh=mesh,
        compiler_params=pltpu.CompilerParams(use_tc_tiling_on_sc=False),
        scratch_types=[
            pltpu.VMEM((nbw, _B), jnp.int32),
            pltpu.VMEM((nbw, _B), jnp.int32),
            pltpu.VMEM((2 * _MAC, _B, _DC), jnp.float32),
            pltpu.VMEM((rbr, _DC), jnp.float32),
            pltpu.VMEM_SHARED((npad, _DC), jnp.float32),
            pltpu.SemaphoreType.DMA,
            pltpu.SemaphoreType.DMA,
        ],
    )
    outs = f(*tables, src3d, dst3d)
    return outs if isinstance(outs, (list, tuple)) else [outs]


def _deg_count(dst3d, n):
    nbw = dst3d.shape[1]
    npad = _npad(n)
    rpt = npad // _NS
    rbr = 128
    assert rpt % rbr == 0

    mesh = plsc.VectorSubcoreMesh(core_axis_name="c", subcore_axis_name="s")

    def body(dst_hbm, out_hbm, didx_v, ones_v, rb_v, acc_sh, ssem):
        c = lax.axis_index("c")
        s = lax.axis_index("s")
        wid = s * _NC + c

        one = jnp.ones((16,), jnp.float32)

        def orow(r, carry):
            ones_v[r, pl.ds(0, 16)] = one
            return carry
        lax.fori_loop(0, _B, orow, 0)

        _zero_fill(rb_v, rbr, _DDEG)
        for i in range(rpt // rbr):
            pltpu.sync_copy(rb_v, acc_sh.at[pl.ds(s * rpt + i * rbr, rbr)])

        pltpu.sync_copy(dst_hbm.at[wid], didx_v)
        plsc.subcore_barrier()

        def macro(m, carry):
            descs = [
                pltpu.async_copy(
                    ones_v, acc_sh.at[didx_v.at[m * _MAC + k]], ssem,
                    add=True)
                for k in range(_MAC)
            ]
            for d in descs:
                d.wait()
            return carry

        lax.fori_loop(0, nbw // _MAC, macro, 0)
        plsc.subcore_barrier()

        for i in range(rpt // rbr):
            pltpu.sync_copy(acc_sh.at[pl.ds(s * rpt + i * rbr, rbr)], rb_v)
            pltpu.sync_copy(
                rb_v, out_hbm.at[pl.ds(c * npad + s * rpt + i * rbr, rbr)])

    f = pl.kernel(
        body,
        out_type=jax.ShapeDtypeStruct((2 * npad, _DDEG), jnp.float32),
        mesh=mesh,
        compiler_params=pltpu.CompilerParams(use_tc_tiling_on_sc=False),
        scratch_types=[
            pltpu.VMEM((nbw, _B), jnp.int32),
            pltpu.VMEM((_B, _DDEG), jnp.float32),
            pltpu.VMEM((rbr, _DDEG), jnp.float32),
            pltpu.VMEM_SHARED((npad, _DDEG), jnp.float32),
            pltpu.SemaphoreType.DMA,
        ],
    )
    return f(dst3d)


def _tc_scale_matmul(cnt0, cnt1, x, w1):
    n = x.shape[0]
    dh = w1.shape[1]

    def body(c0_ref, c1_ref, x_ref, w_ref, lo_ref, hi_ref, dinv_ref):
        cnt = c0_ref[:, 0:1] + c1_ref[:, 0:1]
        dinv = lax.rsqrt(cnt + 1.0)
        h = jnp.dot(x_ref[:, :], w_ref[:, :],
                    preferred_element_type=jnp.float32) * dinv
        lo_ref[:, :] = h[:, :_DC]
        hi_ref[:, :] = h[:, _DC:]
        dinv_ref[:, :] = dinv

    return pl.pallas_call(
        body,
        out_shape=[
            jax.ShapeDtypeStruct((n, _DC), jnp.float32),
            jax.ShapeDtypeStruct((n, dh - _DC), jnp.float32),
            jax.ShapeDtypeStruct((n, 1), jnp.float32),
        ],
    )(cnt0, cnt1, x, w1)


def _tc_bn_relu_matmul(acc_lo, acc_hi, hp_lo, hp_hi, dinv, b1, gamma1,
                       beta1, w2):
    n = hp_lo.shape[0]

    def body(al_ref, ah_ref, hl_ref, hh_ref, dinv_ref, b1_ref, g_ref,
             be_ref, w_ref, out_ref):
        dinv = dinv_ref[:, :]
        zl = dinv * (al_ref[0] + al_ref[1] + hl_ref[:, :])
        zh = dinv * (ah_ref[0] + ah_ref[1] + hh_ref[:, :])
        z = jnp.concatenate([zl, zh], axis=1) + b1_ref[:, :]
        mean = jnp.mean(z, axis=0, keepdims=True)
        zc = z - mean
        var = jnp.mean(zc * zc, axis=0, keepdims=True)
        zn = zc * lax.rsqrt(var + 1e-5) * g_ref[:, :] + be_ref[:, :]
        r = jnp.maximum(zn, 0.0)
        out_ref[:, :] = jnp.dot(
            r, w_ref[:, :], preferred_element_type=jnp.float32) * dinv

    return pl.pallas_call(
        body,
        out_shape=jax.ShapeDtypeStruct((n, w2.shape[1]), jnp.float32),
    )(acc_lo, acc_hi, hp_lo, hp_hi, dinv, b1, gamma1, beta1, w2)


def _tc_finish(acc, h2p, dinv, b2):
    n = h2p.shape[0]

    def body(acc_ref, h2_ref, dinv_ref, b2_ref, out_ref):
        out_ref[:, :] = (dinv_ref[:, :]
                         * (acc_ref[0] + acc_ref[1] + h2_ref[:, :])
                         + b2_ref[:, :])

    return pl.pallas_call(
        body,
        out_shape=jax.ShapeDtypeStruct((n, h2p.shape[1]), jnp.float32),
    )(acc, h2p, dinv, b2)


def kernel(x, edge_index, W1, b1, gamma1, beta1, W2, b2):
    n = x.shape[0]
    e = edge_index.shape[1]
    npad = _npad(n)
    assert npad > n

    grp = _B * _NW * _MAC
    epad = ((e + grp - 1) // grp) * grp
    pad = epad - e
    src_p = jnp.concatenate(
        [edge_index[0], jnp.zeros((pad,), edge_index.dtype)])
    dst_p = jnp.concatenate(
        [edge_index[1],
         n + (jnp.arange(pad, dtype=edge_index.dtype) % (npad - n))])

    src3d = src_p.reshape(_NW, epad // (_B * _NW), _B)
    dst3d = dst_p.reshape(_NW, epad // (_B * _NW), _B)

    deg = _deg_count(dst3d, n).reshape(2, npad, _DDEG)
    hp_lo, hp_hi, dinv = _tc_scale_matmul(deg[0, :n], deg[1, :n], x, W1)

    acc_lo, acc_hi = _seg_sum_rows([hp_lo, hp_hi], src3d, dst3d, n)
    h2p = _tc_bn_relu_matmul(
        acc_lo.reshape(2, npad, _DC)[:, :n],
        acc_hi.reshape(2, npad, _DC)[:, :n],
        hp_lo, hp_hi, dinv,
        b1.reshape(1, -1), gamma1.reshape(1, -1), beta1.reshape(1, -1), W2)

    (acc2,) = _seg_sum_rows([h2p], src3d, dst3d, n)
    out = _tc_finish(acc2.reshape(2, npad, _DC)[:, :n], h2p, dinv,
                     b2.reshape(1, -1))
    return out


# speedup vs baseline: 1.1237x; 1.1237x over previous
import jax
import jax.numpy as jnp
from jax import lax
from jax.experimental import pallas as pl
from jax.experimental.pallas import tpu as pltpu
from jax.experimental.pallas import tpu_sc as plsc

_NC = 2
_NS = 16
_NW = _NC * _NS
_B = 128
_MAC = 2
_DC = 64
_DDEG = 16


def _npad(n):
    return ((n + _NS * 128 - 1) // (_NS * 128)) * (_NS * 128)


def _zero_fill(ref, rows, cols):
    z = jnp.zeros((16,), jnp.float32)

    def row(r, carry):
        def col(k, c2):
            ref[r, pl.ds(k * 16, 16)] = z
            return c2
        return lax.fori_loop(0, cols // 16, col, carry)

    lax.fori_loop(0, rows, row, 0)


def _seg_sum_rows(tables, src3d, dst3d, n):
    nt = len(tables)
    nbw = src3d.shape[1]
    nmac = nbw // _MAC
    npad = _npad(n)
    rpt = npad // _NS
    rbr = 128
    assert nbw % _MAC == 0 and rpt % rbr == 0

    mesh = plsc.VectorSubcoreMesh(core_axis_name="c", subcore_axis_name="s")

    def body(*refs):
        table_hbms = refs[:nt]
        src_hbm, dst_hbm = refs[nt], refs[nt + 1]
        out_hbms = refs[nt + 2:2 * nt + 2]
        sidx_v, didx_v, rows_v, rb_v, acc_sh, gsem, ssem = refs[2 * nt + 2:]

        c = lax.axis_index("c")
        s = lax.axis_index("s")
        wid = s * _NC + c

        _zero_fill(rb_v, rbr, _DC)
        for i in range(rpt // rbr):
            pltpu.sync_copy(rb_v, acc_sh.at[pl.ds(s * rpt + i * rbr, rbr)])

        pltpu.sync_copy(src_hbm.at[wid], sidx_v)
        pltpu.sync_copy(dst_hbm.at[wid], didx_v)
        plsc.subcore_barrier()

        for t in range(nt):
            table_hbm = table_hbms[t]

            for k in range(_MAC):
                pltpu.async_copy(
                    table_hbm.at[sidx_v.at[k]], rows_v.at[k], gsem)

            def macro(m, carry):
                mb = lax.rem(m, 2) * _MAC
                for k in range(_MAC):
                    pltpu.make_async_copy(
                        table_hbm.at[sidx_v.at[m * _MAC + k]],
                        rows_v.at[mb + k], gsem).wait()

                @pl.when(m < nmac - 1)
                def _():
                    for k in range(_MAC):
                        pltpu.async_copy(
                            table_hbm.at[sidx_v.at[(m + 1) * _MAC + k]],
                            rows_v.at[(_MAC - mb) + k], gsem)

                descs = [
                    pltpu.async_copy(
                        rows_v.at[mb + k],
                        acc_sh.at[didx_v.at[m * _MAC + k]], ssem, add=True)
                    for k in range(_MAC)
                ]
                for d in descs:
                    d.wait()
                return carry

            lax.fori_loop(0, nmac, macro, 0)
            plsc.subcore_barrier()

            for i in range(rpt // rbr):
                pltpu.sync_copy(acc_sh.at[pl.ds(s * rpt + i * rbr, rbr)],
                                rb_v)
                pltpu.sync_copy(
                    rb_v,
                    out_hbms[t].at[pl.ds(c * npad + s * rpt + i * rbr, rbr)])
                if t < nt - 1:
                    _zero_fill(rb_v, rbr, _DC)
                    pltpu.sync_copy(
                        rb_v, acc_sh.at[pl.ds(s * rpt + i * rbr, rbr)])
            if t < nt - 1:
                plsc.subcore_barrier()

    f = pl.kernel(
        body,
        out_type=[jax.ShapeDtypeStruct((2 * npad, _DC), jnp.float32)
                  for _ in range(nt)],
        mesh=mesh,
        compiler_params=pltpu.CompilerParams(use_tc_tiling_on_sc=False),
        scratch_types=[
            pltpu.VMEM((nbw, _B), jnp.int32),
            pltpu.VMEM((nbw, _B), jnp.int32),
            pltpu.VMEM((2 * _MAC, _B, _DC), jnp.float32),
            pltpu.VMEM((rbr, _DC), jnp.float32),
            pltpu.VMEM_SHARED((npad, _DC), jnp.float32),
            pltpu.SemaphoreType.DMA,
            pltpu.SemaphoreType.DMA,
        ],
    )
    outs = f(*tables, src3d, dst3d)
    return outs if isinstance(outs, (list, tuple)) else [outs]


def _deg_count(dst3d, n):
    nbw = dst3d.shape[1]
    npad = _npad(n)
    rpt = npad // _NS
    rbr = 128
    assert rpt % rbr == 0

    mesh = plsc.VectorSubcoreMesh(core_axis_name="c", subcore_axis_name="s")

    def body(dst_hbm, out_hbm, didx_v, ones_v, rb_v, acc_sh, ssem):
        c = lax.axis_index("c")
        s = lax.axis_index("s")
        wid = s * _NC + c

        one = jnp.ones((16,), jnp.float32)

        def orow(r, carry):
            ones_v[r, pl.ds(0, 16)] = one
            return carry
        lax.fori_loop(0, _B, orow, 0)

        _zero_fill(rb_v, rbr, _DDEG)
        for i in range(rpt // rbr):
            pltpu.sync_copy(rb_v, acc_sh.at[pl.ds(s * rpt + i * rbr, rbr)])

        pltpu.sync_copy(dst_hbm.at[wid], didx_v)
        plsc.subcore_barrier()

        def macro(m, carry):
            descs = [
                pltpu.async_copy(
                    ones_v, acc_sh.at[didx_v.at[m * _MAC + k]], ssem,
                    add=True)
                for k in range(_MAC)
            ]
            for d in descs:
                d.wait()
            return carry

        lax.fori_loop(0, nbw // _MAC, macro, 0)
        plsc.subcore_barrier()

        for i in range(rpt // rbr):
            pltpu.sync_copy(acc_sh.at[pl.ds(s * rpt + i * rbr, rbr)], rb_v)
            pltpu.sync_copy(
                rb_v, out_hbm.at[pl.ds(c * npad + s * rpt + i * rbr, rbr)])

    f = pl.kernel(
        body,
        out_type=jax.ShapeDtypeStruct((2 * npad, _DDEG), jnp.float32),
        mesh=mesh,
        compiler_params=pltpu.CompilerParams(use_tc_tiling_on_sc=False),
        scratch_types=[
            pltpu.VMEM((nbw, _B), jnp.int32),
            pltpu.VMEM((_B, _DDEG), jnp.float32),
            pltpu.VMEM((rbr, _DDEG), jnp.float32),
            pltpu.VMEM_SHARED((npad, _DDEG), jnp.float32),
            pltpu.SemaphoreType.DMA,
        ],
    )
    return f(dst3d)


def _tc_scale_matmul(cnt0, cnt1, x, w1):
    n = x.shape[0]
    dh = w1.shape[1]

    def body(c0_ref, c1_ref, x_ref, w_ref, lo_ref, hi_ref, dinv_ref):
        cnt = c0_ref[:, 0:1] + c1_ref[:, 0:1]
        dinv = lax.rsqrt(cnt + 1.0)
        h = jnp.dot(x_ref[:, :], w_ref[:, :],
                    preferred_element_type=jnp.float32) * dinv
        lo_ref[:, :] = h[:, :_DC]
        hi_ref[:, :] = h[:, _DC:]
        dinv_ref[:, :] = dinv

    return pl.pallas_call(
        body,
        out_shape=[
            jax.ShapeDtypeStruct((n, _DC), jnp.float32),
            jax.ShapeDtypeStruct((n, dh - _DC), jnp.float32),
            jax.ShapeDtypeStruct((n, 1), jnp.float32),
        ],
    )(cnt0, cnt1, x, w1)


def _tc_bn_relu_matmul(acc_lo, acc_hi, hp_lo, hp_hi, dinv, b1, gamma1,
                       beta1, w2):
    n = hp_lo.shape[0]

    def body(al_ref, ah_ref, hl_ref, hh_ref, dinv_ref, b1_ref, g_ref,
             be_ref, w_ref, out_ref):
        dinv = dinv_ref[:, :]
        zl = dinv * (al_ref[0] + al_ref[1] + hl_ref[:, :])
        zh = dinv * (ah_ref[0] + ah_ref[1] + hh_ref[:, :])
        z = jnp.concatenate([zl, zh], axis=1) + b1_ref[:, :]
        mean = jnp.mean(z, axis=0, keepdims=True)
        zc = z - mean
        var = jnp.mean(zc * zc, axis=0, keepdims=True)
        zn = zc * lax.rsqrt(var + 1e-5) * g_ref[:, :] + be_ref[:, :]
        r = jnp.maximum(zn, 0.0)
        out_ref[:, :] = jnp.dot(
            r, w_ref[:, :], preferred_element_type=jnp.float32) * dinv

    return pl.pallas_call(
        body,
        out_shape=jax.ShapeDtypeStruct((n, w2.shape[1]), jnp.float32),
    )(acc_lo, acc_hi, hp_lo, hp_hi, dinv, b1, gamma1, beta1, w2)


def _tc_finish(acc, h2p, dinv, b2):
    n = h2p.shape[0]

    def body(acc_ref, h2_ref, dinv_ref, b2_ref, out_ref):
        out_ref[:, :] = (dinv_ref[:, :]
                         * (acc_ref[0] + acc_ref[1] + h2_ref[:, :])
                         + b2_ref[:, :])

    return pl.pallas_call(
        body,
        out_shape=jax.ShapeDtypeStruct((n, h2p.shape[1]), jnp.float32),
    )(acc, h2p, dinv, b2)


def kernel(x, edge_index, W1, b1, gamma1, beta1, W2, b2):
    n = x.shape[0]
    e = edge_index.shape[1]
    npad = _npad(n)
    assert npad > n

    assert e % _NW == 0
    rpw = e // _NW
    grp = _B * _MAC
    epw = ((rpw + grp - 1) // grp) * grp
    ppw = epw - rpw
    nbw = epw // _B
    idt = edge_index.dtype
    src_w = edge_index[0].reshape(_NW, rpw)
    dst_w = edge_index[1].reshape(_NW, rpw)
    if ppw:
        pad_src = jnp.zeros((_NW, ppw), idt)
        rot = 8 * jnp.arange(_NW, dtype=idt)[:, None]
        pad_dst = n + (jnp.arange(ppw, dtype=idt)[None, :] + rot) % (npad - n)
        src_w = jnp.concatenate([src_w, pad_src], axis=1)
        dst_w = jnp.concatenate([dst_w, pad_dst], axis=1)
    src3d = src_w.reshape(_NW, nbw, _B)
    dst3d = dst_w.reshape(_NW, nbw, _B)

    deg = _deg_count(dst3d, n).reshape(2, npad, _DDEG)
    hp_lo, hp_hi, dinv = _tc_scale_matmul(deg[0, :n], deg[1, :n], x, W1)

    acc_lo, acc_hi = _seg_sum_rows([hp_lo, hp_hi], src3d, dst3d, n)
    h2p = _tc_bn_relu_matmul(
        acc_lo.reshape(2, npad, _DC)[:, :n],
        acc_hi.reshape(2, npad, _DC)[:, :n],
        hp_lo, hp_hi, dinv,
        b1.reshape(1, -1), gamma1.reshape(1, -1), beta1.reshape(1, -1), W2)

    (acc2,) = _seg_sum_rows([h2p], src3d, dst3d, n)
    out = _tc_finish(acc2.reshape(2, npad, _DC)[:, :n], h2p, dinv,
                     b2.reshape(1, -1))
    return out


# speedup vs baseline: 1.4389x; 1.2805x over previous
import jax
import jax.numpy as jnp
from jax import lax
from jax.experimental import pallas as pl
from jax.experimental.pallas import tpu as pltpu
from jax.experimental.pallas import tpu_sc as plsc

_NC = 2
_NS = 16
_NW = _NC * _NS
_B = 128
_DC = 64
_DDEG = 16


def _npad(n):
    return ((n + _NS * 128 - 1) // (_NS * 128)) * (_NS * 128)


def _zero_fill(ref, rows, cols):
    z = jnp.zeros((16,), jnp.float32)

    def row(r, carry):
        def col(k, c2):
            ref[r, pl.ds(k * 16, 16)] = z
            return c2
        return lax.fori_loop(0, cols // 16, col, carry)

    lax.fori_loop(0, rows, row, 0)


def _seg_sum_rows(tables, src3d, dst3d, n):
    nt = len(tables)
    nbw = src3d.shape[1]
    npad = _npad(n)
    rpt = npad // _NS
    rbr = 128
    assert rpt % rbr == 0

    mesh = plsc.VectorSubcoreMesh(core_axis_name="c", subcore_axis_name="s")

    def body(*refs):
        table_hbms = refs[:nt]
        src_hbm, dst_hbm = refs[nt], refs[nt + 1]
        out_hbms = refs[nt + 2:2 * nt + 2]
        sidx_v, didx_v, rows_v, rb_v, acc_sh, gsem = refs[2 * nt + 2:]

        c = lax.axis_index("c")
        s = lax.axis_index("s")
        wid = s * _NC + c

        _zero_fill(rb_v, rbr, _DC)
        for i in range(rpt // rbr):
            pltpu.sync_copy(rb_v, acc_sh.at[pl.ds(s * rpt + i * rbr, rbr)])

        pltpu.sync_copy(src_hbm.at[wid], sidx_v)
        pltpu.sync_copy(dst_hbm.at[wid], didx_v)
        plsc.subcore_barrier()

        for t in range(nt):
            table_hbm = table_hbms[t]

            pltpu.async_copy(table_hbm.at[sidx_v.at[0]], rows_v.at[0], gsem)

            def step(j, carry):
                b = lax.rem(j, 2)
                pltpu.make_async_copy(
                    table_hbm.at[sidx_v.at[j]], rows_v.at[b], gsem).wait()

                @pl.when(j < nbw - 1)
                def _():
                    pltpu.async_copy(
                        table_hbm.at[sidx_v.at[j + 1]], rows_v.at[1 - b],
                        gsem)

                pltpu.sync_copy(rows_v.at[b], acc_sh.at[didx_v.at[j]],
                                add=True)
                return carry

            lax.fori_loop(0, nbw, step, 0)
            plsc.subcore_barrier()

            for i in range(rpt // rbr):
                pltpu.sync_copy(acc_sh.at[pl.ds(s * rpt + i * rbr, rbr)],
                                rb_v)
                pltpu.sync_copy(
                    rb_v,
                    out_hbms[t].at[pl.ds(c * npad + s * rpt + i * rbr, rbr)])
                if t < nt - 1:
                    _zero_fill(rb_v, rbr, _DC)
                    pltpu.sync_copy(
                        rb_v, acc_sh.at[pl.ds(s * rpt + i * rbr, rbr)])
            if t < nt - 1:
                plsc.subcore_barrier()

    f = pl.kernel(
        body,
        out_type=[jax.ShapeDtypeStruct((2 * npad, _DC), jnp.float32)
                  for _ in range(nt)],
        mesh=mesh,
        compiler_params=pltpu.CompilerParams(use_tc_tiling_on_sc=False),
        scratch_types=[
            pltpu.VMEM((nbw, _B), jnp.int32),
            pltpu.VMEM((nbw, _B), jnp.int32),
            pltpu.VMEM((2, _B, _DC), jnp.float32),
            pltpu.VMEM((rbr, _DC), jnp.float32),
            pltpu.VMEM_SHARED((npad, _DC), jnp.float32),
            pltpu.SemaphoreType.DMA,
        ],
    )
    outs = f(*tables, src3d, dst3d)
    return outs if isinstance(outs, (list, tuple)) else [outs]


def _deg_count(dst3d, n):
    nbw = dst3d.shape[1]
    npad = _npad(n)
    rpt = npad // _NS
    rbr = 128
    assert rpt % rbr == 0

    mesh = plsc.VectorSubcoreMesh(core_axis_name="c", subcore_axis_name="s")

    def body(dst_hbm, out_hbm, didx_v, ones_v, rb_v, acc_sh):
        c = lax.axis_index("c")
        s = lax.axis_index("s")
        wid = s * _NC + c

        one = jnp.ones((16,), jnp.float32)

        def orow(r, carry):
            ones_v[r, pl.ds(0, 16)] = one
            return carry
        lax.fori_loop(0, _B, orow, 0)

        _zero_fill(rb_v, rbr, _DDEG)
        for i in range(rpt // rbr):
            pltpu.sync_copy(rb_v, acc_sh.at[pl.ds(s * rpt + i * rbr, rbr)])

        pltpu.sync_copy(dst_hbm.at[wid], didx_v)
        plsc.subcore_barrier()

        def step(j, carry):
            pltpu.sync_copy(ones_v, acc_sh.at[didx_v.at[j]], add=True)
            return carry

        lax.fori_loop(0, nbw, step, 0)
        plsc.subcore_barrier()

        for i in range(rpt // rbr):
            pltpu.sync_copy(acc_sh.at[pl.ds(s * rpt + i * rbr, rbr)], rb_v)
            pltpu.sync_copy(
                rb_v, out_hbm.at[pl.ds(c * npad + s * rpt + i * rbr, rbr)])

    f = pl.kernel(
        body,
        out_type=jax.ShapeDtypeStruct((2 * npad, _DDEG), jnp.float32),
        mesh=mesh,
        compiler_params=pltpu.CompilerParams(use_tc_tiling_on_sc=False),
        scratch_types=[
            pltpu.VMEM((nbw, _B), jnp.int32),
            pltpu.VMEM((_B, _DDEG), jnp.float32),
            pltpu.VMEM((rbr, _DDEG), jnp.float32),
            pltpu.VMEM_SHARED((npad, _DDEG), jnp.float32),
        ],
    )
    return f(dst3d)


def _tc_scale_matmul(cnt0, cnt1, x, w1):
    n = x.shape[0]
    dh = w1.shape[1]

    def body(c0_ref, c1_ref, x_ref, w_ref, lo_ref, hi_ref, dinv_ref):
        cnt = c0_ref[:, 0:1] + c1_ref[:, 0:1]
        dinv = lax.rsqrt(cnt + 1.0)
        h = jnp.dot(x_ref[:, :], w_ref[:, :],
                    preferred_element_type=jnp.float32) * dinv
        lo_ref[:, :] = h[:, :_DC]
        hi_ref[:, :] = h[:, _DC:]
        dinv_ref[:, :] = dinv

    return pl.pallas_call(
        body,
        out_shape=[
            jax.ShapeDtypeStruct((n, _DC), jnp.float32),
            jax.ShapeDtypeStruct((n, dh - _DC), jnp.float32),
            jax.ShapeDtypeStruct((n, 1), jnp.float32),
        ],
    )(cnt0, cnt1, x, w1)


def _tc_bn_relu_matmul(acc_lo, acc_hi, hp_lo, hp_hi, dinv, b1, gamma1,
                       beta1, w2):
    n = hp_lo.shape[0]

    def body(al_ref, ah_ref, hl_ref, hh_ref, dinv_ref, b1_ref, g_ref,
             be_ref, w_ref, out_ref):
        dinv = dinv_ref[:, :]
        zl = dinv * (al_ref[0] + al_ref[1] + hl_ref[:, :])
        zh = dinv * (ah_ref[0] + ah_ref[1] + hh_ref[:, :])
        z = jnp.concatenate([zl, zh], axis=1) + b1_ref[:, :]
        mean = jnp.mean(z, axis=0, keepdims=True)
        zc = z - mean
        var = jnp.mean(zc * zc, axis=0, keepdims=True)
        zn = zc * lax.rsqrt(var + 1e-5) * g_ref[:, :] + be_ref[:, :]
        r = jnp.maximum(zn, 0.0)
        out_ref[:, :] = jnp.dot(
            r, w_ref[:, :], preferred_element_type=jnp.float32) * dinv

    return pl.pallas_call(
        body,
        out_shape=jax.ShapeDtypeStruct((n, w2.shape[1]), jnp.float32),
    )(acc_lo, acc_hi, hp_lo, hp_hi, dinv, b1, gamma1, beta1, w2)


def _tc_finish(acc, h2p, dinv, b2):
    n = h2p.shape[0]

    def body(acc_ref, h2_ref, dinv_ref, b2_ref, out_ref):
        out_ref[:, :] = (dinv_ref[:, :]
                         * (acc_ref[0] + acc_ref[1] + h2_ref[:, :])
                         + b2_ref[:, :])

    return pl.pallas_call(
        body,
        out_shape=jax.ShapeDtypeStruct((n, h2p.shape[1]), jnp.float32),
    )(acc, h2p, dinv, b2)


def kernel(x, edge_index, W1, b1, gamma1, beta1, W2, b2):
    n = x.shape[0]
    e = edge_index.shape[1]
    npad = _npad(n)
    assert npad > n and e % _NW == 0

    rpw = e // _NW
    epw = ((rpw + _B - 1) // _B) * _B
    ppw = epw - rpw
    nbw = epw // _B
    idt = edge_index.dtype
    src_w = edge_index[0].reshape(_NW, rpw)
    dst_w = edge_index[1].reshape(_NW, rpw)
    if ppw:
        pad_src = jnp.zeros((_NW, ppw), idt)
        rot = 8 * jnp.arange(_NW, dtype=idt)[:, None]
        pad_dst = n + (jnp.arange(ppw, dtype=idt)[None, :] + rot) % (npad - n)
        src_w = jnp.concatenate([src_w, pad_src], axis=1)
        dst_w = jnp.concatenate([dst_w, pad_dst], axis=1)
    src3d = src_w.reshape(_NW, nbw, _B)
    dst3d = dst_w.reshape(_NW, nbw, _B)

    deg = _deg_count(dst3d, n).reshape(2, npad, _DDEG)
    hp_lo, hp_hi, dinv = _tc_scale_matmul(deg[0, :n], deg[1, :n], x, W1)

    acc_lo, acc_hi = _seg_sum_rows([hp_lo, hp_hi], src3d, dst3d, n)
    h2p = _tc_bn_relu_matmul(
        acc_lo.reshape(2, npad, _DC)[:, :n], acc_hi.reshape(2, npad, _DC)[:, :n],
        hp_lo, hp_hi, dinv,
        b1.reshape(1, -1), gamma1.reshape(1, -1), beta1.reshape(1, -1), W2)

    (acc2,) = _seg_sum_rows([h2p], src3d, dst3d, n)
    out = _tc_finish(acc2.reshape(2, npad, _DC)[:, :n], h2p, dinv,
                     b2.reshape(1, -1))
    return out


# speedup vs baseline: 1.7599x; 1.2231x over previous
import jax
import jax.numpy as jnp
from jax import lax
from jax.experimental import pallas as pl
from jax.experimental.pallas import tpu as pltpu
from jax.experimental.pallas import tpu_sc as plsc

_NC = 2
_NS = 16
_NW = _NC * _NS
_B = 80
_DC = 64
_DDEG = 16


def _npad(n):
    return ((n + _NS * 128 - 1) // (_NS * 128)) * (_NS * 128)


def _zero_fill(ref, rows, cols):
    z = jnp.zeros((16,), jnp.float32)

    def row(r, carry):
        def col(k, c2):
            ref[r, pl.ds(k * 16, 16)] = z
            return c2
        return lax.fori_loop(0, cols // 16, col, carry)

    lax.fori_loop(0, rows, row, 0)


def _seg_sum_rows(tables, src3d, dst3d, n):
    nt = len(tables)
    nbw = src3d.shape[1]
    npad = _npad(n)
    rpt = npad // _NS
    rbr = 128
    assert rpt % rbr == 0

    mesh = plsc.VectorSubcoreMesh(core_axis_name="c", subcore_axis_name="s")

    def body(*refs):
        table_hbms = refs[:nt]
        src_hbm, dst_hbm = refs[nt], refs[nt + 1]
        out_hbms = refs[nt + 2:2 * nt + 2]
        sidx_v, didx_v, rows_v, rb_v, acc_sh = refs[2 * nt + 2:2 * nt + 7]
        gsems = refs[2 * nt + 7:]

        c = lax.axis_index("c")
        s = lax.axis_index("s")
        wid = s * _NC + c

        _zero_fill(rb_v, rbr, _DC)
        for i in range(rpt // rbr):
            pltpu.sync_copy(rb_v, acc_sh.at[pl.ds(s * rpt + i * rbr, rbr)])

        pltpu.sync_copy(src_hbm.at[wid], sidx_v)
        pltpu.sync_copy(dst_hbm.at[wid], didx_v)
        plsc.subcore_barrier()

        for t in range(nt):
            table_hbm = table_hbms[t]

            pltpu.async_copy(table_hbm.at[sidx_v.at[0]], rows_v.at[0],
                             gsems[0])
            pltpu.async_copy(table_hbm.at[sidx_v.at[1]], rows_v.at[1],
                             gsems[1])

            def step2(m, carry):
                for k in range(2):
                    j = 2 * m + k
                    b = lax.rem(j, 4)
                    pltpu.make_async_copy(
                        table_hbm.at[sidx_v.at[j]], rows_v.at[b],
                        gsems[k]).wait()

                    @pl.when(j + 2 < nbw)
                    def _():
                        pltpu.async_copy(
                            table_hbm.at[sidx_v.at[j + 2]],
                            rows_v.at[lax.rem(j + 2, 4)], gsems[k])

                    pltpu.sync_copy(rows_v.at[b], acc_sh.at[didx_v.at[j]],
                                    add=True)
                return carry

            lax.fori_loop(0, nbw // 2, step2, 0)
            plsc.subcore_barrier()

            for i in range(rpt // rbr):
                pltpu.sync_copy(acc_sh.at[pl.ds(s * rpt + i * rbr, rbr)],
                                rb_v)
                pltpu.sync_copy(
                    rb_v,
                    out_hbms[t].at[pl.ds(c * npad + s * rpt + i * rbr, rbr)])
                if t < nt - 1:
                    _zero_fill(rb_v, rbr, _DC)
                    pltpu.sync_copy(
                        rb_v, acc_sh.at[pl.ds(s * rpt + i * rbr, rbr)])
            if t < nt - 1:
                plsc.subcore_barrier()

    f = pl.kernel(
        body,
        out_type=[jax.ShapeDtypeStruct((2 * npad, _DC), jnp.float32)
                  for _ in range(nt)],
        mesh=mesh,
        compiler_params=pltpu.CompilerParams(use_tc_tiling_on_sc=False),
        scratch_types=[
            pltpu.VMEM((nbw, _B), jnp.int32),
            pltpu.VMEM((nbw, _B), jnp.int32),
            pltpu.VMEM((4, _B, _DC), jnp.float32),
            pltpu.VMEM((rbr, _DC), jnp.float32),
            pltpu.VMEM_SHARED((npad, _DC), jnp.float32),
            pltpu.SemaphoreType.DMA,
            pltpu.SemaphoreType.DMA,
        ],
    )
    outs = f(*tables, src3d, dst3d)
    return outs if isinstance(outs, (list, tuple)) else [outs]


def _deg_count(dst3d, n):
    nbw = dst3d.shape[1]
    npad = _npad(n)
    rpt = npad // _NS
    rbr = 128
    assert rpt % rbr == 0

    mesh = plsc.VectorSubcoreMesh(core_axis_name="c", subcore_axis_name="s")

    def body(dst_hbm, out_hbm, didx_v, ones_v, rb_v, acc_sh):
        c = lax.axis_index("c")
        s = lax.axis_index("s")
        wid = s * _NC + c

        one = jnp.ones((16,), jnp.float32)

        def orow(r, carry):
            ones_v[r, pl.ds(0, 16)] = one
            return carry
        lax.fori_loop(0, _B, orow, 0)

        _zero_fill(rb_v, rbr, _DDEG)
        for i in range(rpt // rbr):
            pltpu.sync_copy(rb_v, acc_sh.at[pl.ds(s * rpt + i * rbr, rbr)])

        pltpu.sync_copy(dst_hbm.at[wid], didx_v)
        plsc.subcore_barrier()

        def step(j, carry):
            pltpu.sync_copy(ones_v, acc_sh.at[didx_v.at[j]], add=True)
            return carry

        lax.fori_loop(0, nbw, step, 0)
        plsc.subcore_barrier()

        for i in range(rpt // rbr):
            pltpu.sync_copy(acc_sh.at[pl.ds(s * rpt + i * rbr, rbr)], rb_v)
            pltpu.sync_copy(
                rb_v, out_hbm.at[pl.ds(c * npad + s * rpt + i * rbr, rbr)])

    f = pl.kernel(
        body,
        out_type=jax.ShapeDtypeStruct((2 * npad, _DDEG), jnp.float32),
        mesh=mesh,
        compiler_params=pltpu.CompilerParams(use_tc_tiling_on_sc=False),
        scratch_types=[
            pltpu.VMEM((nbw, _B), jnp.int32),
            pltpu.VMEM((_B, _DDEG), jnp.float32),
            pltpu.VMEM((rbr, _DDEG), jnp.float32),
            pltpu.VMEM_SHARED((npad, _DDEG), jnp.float32),
        ],
    )
    return f(dst3d)


def _tc_scale_matmul(cnt0, cnt1, x, w1):
    n = x.shape[0]
    dh = w1.shape[1]

    def body(c0_ref, c1_ref, x_ref, w_ref, lo_ref, hi_ref, dinv_ref):
        cnt = c0_ref[:, 0:1] + c1_ref[:, 0:1]
        dinv = lax.rsqrt(cnt + 1.0)
        h = jnp.dot(x_ref[:, :], w_ref[:, :],
                    preferred_element_type=jnp.float32) * dinv
        lo_ref[:, :] = h[:, :_DC]
        hi_ref[:, :] = h[:, _DC:]
        dinv_ref[:, :] = dinv

    return pl.pallas_call(
        body,
        out_shape=[
            jax.ShapeDtypeStruct((n, _DC), jnp.float32),
            jax.ShapeDtypeStruct((n, dh - _DC), jnp.float32),
            jax.ShapeDtypeStruct((n, 1), jnp.float32),
        ],
    )(cnt0, cnt1, x, w1)


def _tc_bn_relu_matmul(acc_lo, acc_hi, hp_lo, hp_hi, dinv, b1, gamma1,
                       beta1, w2):
    n = hp_lo.shape[0]

    def body(al_ref, ah_ref, hl_ref, hh_ref, dinv_ref, b1_ref, g_ref,
             be_ref, w_ref, out_ref):
        dinv = dinv_ref[:, :]
        zl = dinv * (al_ref[0] + al_ref[1] + hl_ref[:, :])
        zh = dinv * (ah_ref[0] + ah_ref[1] + hh_ref[:, :])
        z = jnp.concatenate([zl, zh], axis=1) + b1_ref[:, :]
        mean = jnp.mean(z, axis=0, keepdims=True)
        zc = z - mean
        var = jnp.mean(zc * zc, axis=0, keepdims=True)
        zn = zc * lax.rsqrt(var + 1e-5) * g_ref[:, :] + be_ref[:, :]
        r = jnp.maximum(zn, 0.0)
        out_ref[:, :] = jnp.dot(
            r, w_ref[:, :], preferred_element_type=jnp.float32) * dinv

    return pl.pallas_call(
        body,
        out_shape=jax.ShapeDtypeStruct((n, w2.shape[1]), jnp.float32),
    )(acc_lo, acc_hi, hp_lo, hp_hi, dinv, b1, gamma1, beta1, w2)


def _tc_finish(acc, h2p, dinv, b2):
    n = h2p.shape[0]

    def body(acc_ref, h2_ref, dinv_ref, b2_ref, out_ref):
        out_ref[:, :] = (dinv_ref[:, :]
                         * (acc_ref[0] + acc_ref[1] + h2_ref[:, :])
                         + b2_ref[:, :])

    return pl.pallas_call(
        body,
        out_shape=jax.ShapeDtypeStruct((n, h2p.shape[1]), jnp.float32),
    )(acc, h2p, dinv, b2)


def kernel(x, edge_index, W1, b1, gamma1, beta1, W2, b2):
    n = x.shape[0]
    e = edge_index.shape[1]
    npad = _npad(n)
    assert npad > n and e % _NW == 0

    rpw = e // _NW
    epw = ((rpw + 2 * _B - 1) // (2 * _B)) * (2 * _B)
    ppw = epw - rpw
    nbw = epw // _B
    idt = edge_index.dtype
    src_w = edge_index[0].reshape(_NW, rpw)
    dst_w = edge_index[1].reshape(_NW, rpw)
    if ppw:
        pad_src = jnp.zeros((_NW, ppw), idt)
        rot = 8 * jnp.arange(_NW, dtype=idt)[:, None]
        pad_dst = n + (jnp.arange(ppw, dtype=idt)[None, :] + rot) % (npad - n)
        src_w = jnp.concatenate([src_w, pad_src], axis=1)
        dst_w = jnp.concatenate([dst_w, pad_dst], axis=1)
    src3d = src_w.reshape(_NW, nbw, _B)
    dst3d = dst_w.reshape(_NW, nbw, _B)

    deg = _deg_count(dst3d, n).reshape(2, npad, _DDEG)
    hp_lo, hp_hi, dinv = _tc_scale_matmul(deg[0, :n], deg[1, :n], x, W1)

    acc_lo, acc_hi = _seg_sum_rows([hp_lo, hp_hi], src3d, dst3d, n)
    h2p = _tc_bn_relu_matmul(
        acc_lo.reshape(2, npad, _DC)[:, :n], acc_hi.reshape(2, npad, _DC)[:, :n],
        hp_lo, hp_hi, dinv,
        b1.reshape(1, -1), gamma1.reshape(1, -1), beta1.reshape(1, -1), W2)

    (acc2,) = _seg_sum_rows([h2p], src3d, dst3d, n)
    out = _tc_finish(acc2.reshape(2, npad, _DC)[:, :n], h2p, dinv,
                     b2.reshape(1, -1))
    return out
